# baseline (speedup 1.0000x reference)
import functools

import jax
import jax.numpy as jnp
from jax import lax
from jax.experimental import pallas as pl
from jax.experimental.pallas import tpu as pltpu
from jax.experimental.pallas import tpu_sc as plsc

MAXLEN = 3
EMBED_DIM = 640
LANES = 16
NCHUNK = EMBED_DIM // LANES

NC = 2
NS = 16
NW = NC * NS


def _add_kernel(x_ref, pos_ref, o_ref):
    o_ref[...] = x_ref[...] + pos_ref[...]


def _tc_add(xt, pos3, blk=1024):
    n = xt.shape[1]
    return pl.pallas_call(
        _add_kernel,
        grid=(n // blk,),
        in_specs=[
            pl.BlockSpec((MAXLEN, blk, EMBED_DIM), lambda i: (0, i, 0)),
            pl.BlockSpec((MAXLEN, 1, EMBED_DIM), lambda i: (0, 0, 0)),
        ],
        out_specs=pl.BlockSpec((MAXLEN, blk, EMBED_DIM), lambda i: (0, i, 0)),
        out_shape=jax.ShapeDtypeStruct((MAXLEN, n, EMBED_DIM), xt.dtype),
        compiler_params=pltpu.CompilerParams(
            dimension_semantics=("arbitrary",),
        ),
    )(xt, pos3)


def _sc_add(xt, pos_table, rows_b=64):
    n = xt.shape[1]
    per_w = n // NW
    rounds = per_w // rows_b
    mesh = plsc.VectorSubcoreMesh(core_axis_name="c", subcore_axis_name="s")

    @functools.partial(
        pl.kernel,
        out_type=jax.ShapeDtypeStruct((MAXLEN, n, EMBED_DIM), jnp.float32),
        mesh=mesh,
        scratch_types=[
            pltpu.VMEM((rows_b, EMBED_DIM), jnp.float32),
            pltpu.VMEM((MAXLEN, EMBED_DIM), jnp.float32),
        ],
        compiler_params=pltpu.CompilerParams(use_tc_tiling_on_sc=True),
    )
    def k(xt_hbm, pos_hbm, out_hbm, buf, posv):
        wid = lax.axis_index("s") * NC + lax.axis_index("c")
        base0 = wid * per_w
        pltpu.sync_copy(pos_hbm, posv)
        for p in range(MAXLEN):
            def round_body(b, _, p=p):
                base = base0 + b * rows_b
                pltpu.sync_copy(xt_hbm.at[p, pl.ds(base, rows_b), :], buf)

                def row_body(r, _, p=p):
                    for j in range(NCHUNK):
                        sl = pl.ds(j * LANES, LANES)
                        buf[r, sl] = buf[r, sl] + posv[p, sl]
                    return 0

                lax.fori_loop(0, rows_b, row_body, 0)
                pltpu.sync_copy(buf, out_hbm.at[p, pl.ds(base, rows_b), :])
                return 0

            lax.fori_loop(0, rounds, round_body, 0)

    return k(xt, pos_table)


def kernel(x, pos_table):
    n = x.shape[0]
    xt = jnp.transpose(x, (1, 0, 2))
    out = _sc_add(xt, pos_table)
    return jnp.transpose(out, (1, 0, 2))


# speedup vs baseline: 2.2939x; 2.2939x over previous
import functools

import jax
import jax.numpy as jnp
from jax import lax
from jax.experimental import pallas as pl
from jax.experimental.pallas import tpu as pltpu
from jax.experimental.pallas import tpu_sc as plsc

MAXLEN = 3
EMBED_DIM = 640
LANES = 16
NCHUNK = EMBED_DIM // LANES

NC = 2
NS = 16
NW = NC * NS


def _add_kernel(x_ref, pos_ref, o_ref):
    o_ref[...] = x_ref[...] + pos_ref[...]


def _tc_add(xt, pos3, blk=1024):
    n = xt.shape[1]
    return pl.pallas_call(
        _add_kernel,
        grid=(n // blk,),
        in_specs=[
            pl.BlockSpec((MAXLEN, blk, EMBED_DIM), lambda i: (0, i, 0)),
            pl.BlockSpec((MAXLEN, 1, EMBED_DIM), lambda i: (0, 0, 0)),
        ],
        out_specs=pl.BlockSpec((MAXLEN, blk, EMBED_DIM), lambda i: (0, i, 0)),
        out_shape=jax.ShapeDtypeStruct((MAXLEN, n, EMBED_DIM), xt.dtype),
        compiler_params=pltpu.CompilerParams(
            dimension_semantics=("arbitrary",),
        ),
    )(xt, pos3)


def _sc_add(xt, pos_table, row0, n_sc, rows_b=64):
    per_w = n_sc // NW
    rounds = per_w // rows_b
    mesh = plsc.VectorSubcoreMesh(core_axis_name="c", subcore_axis_name="s")

    @functools.partial(
        pl.kernel,
        out_type=jax.ShapeDtypeStruct((MAXLEN, n_sc, EMBED_DIM), jnp.float32),
        mesh=mesh,
        scratch_types=[
            pltpu.VMEM((rows_b, EMBED_DIM), jnp.float32),
            pltpu.VMEM((MAXLEN, EMBED_DIM), jnp.float32),
        ],
        compiler_params=pltpu.CompilerParams(use_tc_tiling_on_sc=True),
    )
    def k(xt_hbm, pos_hbm, out_hbm, buf, posv):
        wid = lax.axis_index("s") * NC + lax.axis_index("c")
        base0 = wid * per_w
        pltpu.sync_copy(pos_hbm, posv)
        for p in range(MAXLEN):
            def round_body(b, _, p=p):
                base = base0 + b * rows_b
                pltpu.sync_copy(xt_hbm.at[p, pl.ds(row0 + base, rows_b), :], buf)

                for j in range(NCHUNK):
                    sl = pl.ds(j * LANES, LANES)
                    pv = posv[p, sl]

                    def col_body(r, _, sl=sl, pv=pv):
                        buf[r, sl] = buf[r, sl] + pv
                        return 0

                    lax.fori_loop(0, rows_b, col_body, 0, unroll=8)

                pltpu.sync_copy(buf, out_hbm.at[p, pl.ds(base, rows_b), :])
                return 0

            lax.fori_loop(0, rounds, round_body, 0)

    return k(xt, pos_table)


def kernel(x, pos_table):
    n = x.shape[0]
    n_sc = 2048
    n_tc = n - n_sc
    xt = jnp.transpose(x, (1, 0, 2))
    pos3 = pos_table.reshape(MAXLEN, 1, EMBED_DIM)
    sc_out = _sc_add(xt, pos_table, n_tc, n_sc)
    tc_out = pl.pallas_call(
        _add_kernel,
        grid=(n_tc // 1024,),
        in_specs=[
            pl.BlockSpec((MAXLEN, 1024, EMBED_DIM), lambda i: (0, i, 0)),
            pl.BlockSpec((MAXLEN, 1, EMBED_DIM), lambda i: (0, 0, 0)),
        ],
        out_specs=pl.BlockSpec((MAXLEN, 1024, EMBED_DIM), lambda i: (0, i, 0)),
        out_shape=jax.ShapeDtypeStruct((MAXLEN, n_tc, EMBED_DIM), x.dtype),
        compiler_params=pltpu.CompilerParams(
            dimension_semantics=("arbitrary",),
        ),
    )(xt, pos3)
    out = jnp.concatenate([tc_out, sc_out], axis=1)
    return jnp.transpose(out, (1, 0, 2))


# speedup vs baseline: 4.9427x; 2.1548x over previous
import functools

import jax
import jax.numpy as jnp
from jax import lax
from jax.experimental import pallas as pl
from jax.experimental.pallas import tpu as pltpu
from jax.experimental.pallas import tpu_sc as plsc

MAXLEN = 3
EMBED_DIM = 640
LANES = 16
NCHUNK = EMBED_DIM // LANES

NC = 2
NS = 16
NW = NC * NS


def _add_kernel(x_ref, pos_ref, o_ref):
    o_ref[...] = x_ref[...] + pos_ref[...]


def _tc_add(xt, pos3, blk=1024):
    n = xt.shape[1]
    return pl.pallas_call(
        _add_kernel,
        grid=(n // blk,),
        in_specs=[
            pl.BlockSpec((MAXLEN, blk, EMBED_DIM), lambda i: (0, i, 0)),
            pl.BlockSpec((MAXLEN, 1, EMBED_DIM), lambda i: (0, 0, 0)),
        ],
        out_specs=pl.BlockSpec((MAXLEN, blk, EMBED_DIM), lambda i: (0, i, 0)),
        out_shape=jax.ShapeDtypeStruct((MAXLEN, n, EMBED_DIM), xt.dtype),
        compiler_params=pltpu.CompilerParams(
            dimension_semantics=("arbitrary",),
        ),
    )(xt, pos3)


def _sc_add(xt, pos_table, row0, n_sc, rows_b=64):
    per_w = n_sc // NW
    rounds = per_w // rows_b
    mesh = plsc.VectorSubcoreMesh(core_axis_name="c", subcore_axis_name="s")

    @functools.partial(
        pl.kernel,
        out_type=jax.ShapeDtypeStruct((MAXLEN, n_sc, EMBED_DIM), jnp.float32),
        mesh=mesh,
        scratch_types=[
            pltpu.VMEM((rows_b, EMBED_DIM), jnp.float32),
            pltpu.VMEM((MAXLEN, EMBED_DIM), jnp.float32),
        ],
        compiler_params=pltpu.CompilerParams(use_tc_tiling_on_sc=True),
    )
    def k(xt_hbm, pos_hbm, out_hbm, buf, posv):
        wid = lax.axis_index("s") * NC + lax.axis_index("c")
        base0 = wid * per_w
        pltpu.sync_copy(pos_hbm, posv)
        for p in range(MAXLEN):
            def round_body(b, _, p=p):
                base = base0 + b * rows_b
                pltpu.sync_copy(xt_hbm.at[p, pl.ds(row0 + base, rows_b), :], buf)

                for j in range(NCHUNK):
                    sl = pl.ds(j * LANES, LANES)
                    pv = posv[p, sl]

                    def col_body(r, _, sl=sl, pv=pv):
                        buf[r, sl] = buf[r, sl] + pv
                        return 0

                    lax.fori_loop(0, rows_b, col_body, 0, unroll=8)

                pltpu.sync_copy(buf, out_hbm.at[p, pl.ds(base, rows_b), :])
                return 0

            lax.fori_loop(0, rounds, round_body, 0)

    return k(xt, pos_table)


def kernel(x, pos_table):
    n = x.shape[0]
    blk = 512
    xt = jnp.transpose(x, (1, 0, 2))
    pos3 = pos_table.reshape(MAXLEN, 1, EMBED_DIM)
    out = pl.pallas_call(
        _add_kernel,
        grid=(n // blk,),
        in_specs=[
            pl.BlockSpec((MAXLEN, blk, EMBED_DIM), lambda i: (0, i, 0)),
            pl.BlockSpec((MAXLEN, 1, EMBED_DIM), lambda i: (0, 0, 0)),
        ],
        out_specs=pl.BlockSpec((MAXLEN, blk, EMBED_DIM), lambda i: (0, i, 0)),
        out_shape=jax.ShapeDtypeStruct((MAXLEN, n, EMBED_DIM), x.dtype),
        compiler_params=pltpu.CompilerParams(
            dimension_semantics=("arbitrary",),
        ),
    )(xt, pos3)
    return jnp.transpose(out, (1, 0, 2))


# speedup vs baseline: 5.1167x; 1.0352x over previous
import functools

import jax
import jax.numpy as jnp
from jax import lax
from jax.experimental import pallas as pl
from jax.experimental.pallas import tpu as pltpu
from jax.experimental.pallas import tpu_sc as plsc

MAXLEN = 3
EMBED_DIM = 640
LANES = 16
NCHUNK = EMBED_DIM // LANES

NC = 2
NS = 16
NW = NC * NS


def _add_kernel(x_ref, pos_ref, o_ref):
    o_ref[...] = x_ref[...] + pos_ref[...]


def _tc_add(xt, pos3, blk=1024):
    n = xt.shape[1]
    return pl.pallas_call(
        _add_kernel,
        grid=(n // blk,),
        in_specs=[
            pl.BlockSpec((MAXLEN, blk, EMBED_DIM), lambda i: (0, i, 0)),
            pl.BlockSpec((MAXLEN, 1, EMBED_DIM), lambda i: (0, 0, 0)),
        ],
        out_specs=pl.BlockSpec((MAXLEN, blk, EMBED_DIM), lambda i: (0, i, 0)),
        out_shape=jax.ShapeDtypeStruct((MAXLEN, n, EMBED_DIM), xt.dtype),
        compiler_params=pltpu.CompilerParams(
            dimension_semantics=("arbitrary",),
        ),
    )(xt, pos3)


def _sc_add(xt, pos_table, row0, n_sc, rows_b=64):
    per_w = n_sc // NW
    rounds = per_w // rows_b
    mesh = plsc.VectorSubcoreMesh(core_axis_name="c", subcore_axis_name="s")

    @functools.partial(
        pl.kernel,
        out_type=jax.ShapeDtypeStruct((MAXLEN, n_sc, EMBED_DIM), jnp.float32),
        mesh=mesh,
        scratch_types=[
            pltpu.VMEM((rows_b, EMBED_DIM), jnp.float32),
            pltpu.VMEM((MAXLEN, EMBED_DIM), jnp.float32),
        ],
        compiler_params=pltpu.CompilerParams(use_tc_tiling_on_sc=True),
    )
    def k(xt_hbm, pos_hbm, out_hbm, buf, posv):
        wid = lax.axis_index("s") * NC + lax.axis_index("c")
        base0 = wid * per_w
        pltpu.sync_copy(pos_hbm, posv)
        for p in range(MAXLEN):
            def round_body(b, _, p=p):
                base = base0 + b * rows_b
                pltpu.sync_copy(xt_hbm.at[p, pl.ds(row0 + base, rows_b), :], buf)

                for j in range(NCHUNK):
                    sl = pl.ds(j * LANES, LANES)
                    pv = posv[p, sl]

                    def col_body(r, _, sl=sl, pv=pv):
                        buf[r, sl] = buf[r, sl] + pv
                        return 0

                    lax.fori_loop(0, rows_b, col_body, 0, unroll=8)

                pltpu.sync_copy(buf, out_hbm.at[p, pl.ds(base, rows_b), :])
                return 0

            lax.fori_loop(0, rounds, round_body, 0)

    return k(xt, pos_table)


def kernel(x, pos_table):
    n = x.shape[0]
    blk = 2048
    xt = jnp.transpose(x, (1, 0, 2))
    pos3 = pos_table.reshape(MAXLEN, 1, EMBED_DIM)
    out = pl.pallas_call(
        _add_kernel,
        grid=(n // blk,),
        in_specs=[
            pl.BlockSpec((MAXLEN, blk, EMBED_DIM), lambda i: (0, i, 0)),
            pl.BlockSpec((MAXLEN, 1, EMBED_DIM), lambda i: (0, 0, 0)),
        ],
        out_specs=pl.BlockSpec((MAXLEN, blk, EMBED_DIM), lambda i: (0, i, 0)),
        out_shape=jax.ShapeDtypeStruct((MAXLEN, n, EMBED_DIM), x.dtype),
        compiler_params=pltpu.CompilerParams(
            dimension_semantics=("arbitrary",),
            vmem_limit_bytes=100 * 1024 * 1024,
        ),
    )(xt, pos3)
    return jnp.transpose(out, (1, 0, 2))


# speedup vs baseline: 5.2102x; 1.0183x over previous
import functools

import jax
import jax.numpy as jnp
from jax import lax
from jax.experimental import pallas as pl
from jax.experimental.pallas import tpu as pltpu
from jax.experimental.pallas import tpu_sc as plsc

MAXLEN = 3
EMBED_DIM = 640
LANES = 16
NCHUNK = EMBED_DIM // LANES

NC = 2
NS = 16
NW = NC * NS


def _add_kernel(x_ref, pos_ref, o_ref):
    o_ref[...] = x_ref[...] + pos_ref[...]


def _add_kernel_2d(x_ref, pos_ref, o_ref):
    o_ref[...] = x_ref[...] + pos_ref[...][:, None, :]


def _tc_add(xt, pos3, blk=1024):
    n = xt.shape[1]
    return pl.pallas_call(
        _add_kernel,
        grid=(n // blk,),
        in_specs=[
            pl.BlockSpec((MAXLEN, blk, EMBED_DIM), lambda i: (0, i, 0)),
            pl.BlockSpec((MAXLEN, 1, EMBED_DIM), lambda i: (0, 0, 0)),
        ],
        out_specs=pl.BlockSpec((MAXLEN, blk, EMBED_DIM), lambda i: (0, i, 0)),
        out_shape=jax.ShapeDtypeStruct((MAXLEN, n, EMBED_DIM), xt.dtype),
        compiler_params=pltpu.CompilerParams(
            dimension_semantics=("arbitrary",),
        ),
    )(xt, pos3)


def _sc_add(xt, pos_table, row0, n_sc, rows_b=64):
    per_w = n_sc // NW
    rounds = per_w // rows_b
    mesh = plsc.VectorSubcoreMesh(core_axis_name="c", subcore_axis_name="s")

    @functools.partial(
        pl.kernel,
        out_type=jax.ShapeDtypeStruct((MAXLEN, n_sc, EMBED_DIM), jnp.float32),
        mesh=mesh,
        scratch_types=[
            pltpu.VMEM((rows_b, EMBED_DIM), jnp.float32),
            pltpu.VMEM((MAXLEN, EMBED_DIM), jnp.float32),
        ],
        compiler_params=pltpu.CompilerParams(use_tc_tiling_on_sc=True),
    )
    def k(xt_hbm, pos_hbm, out_hbm, buf, posv):
        wid = lax.axis_index("s") * NC + lax.axis_index("c")
        base0 = wid * per_w
        pltpu.sync_copy(pos_hbm, posv)
        for p in range(MAXLEN):
            def round_body(b, _, p=p):
                base = base0 + b * rows_b
                pltpu.sync_copy(xt_hbm.at[p, pl.ds(row0 + base, rows_b), :], buf)

                for j in range(NCHUNK):
                    sl = pl.ds(j * LANES, LANES)
                    pv = posv[p, sl]

                    def col_body(r, _, sl=sl, pv=pv):
                        buf[r, sl] = buf[r, sl] + pv
                        return 0

                    lax.fori_loop(0, rows_b, col_body, 0, unroll=8)

                pltpu.sync_copy(buf, out_hbm.at[p, pl.ds(base, rows_b), :])
                return 0

            lax.fori_loop(0, rounds, round_body, 0)

    return k(xt, pos_table)


def kernel(x, pos_table):
    n = x.shape[0]
    blk = 2048
    xt = jnp.transpose(x, (1, 0, 2))
    out = pl.pallas_call(
        _add_kernel_2d,
        grid=(n // blk,),
        in_specs=[
            pl.BlockSpec((MAXLEN, blk, EMBED_DIM), lambda i: (0, i, 0)),
            pl.BlockSpec((MAXLEN, EMBED_DIM), lambda i: (0, 0)),
        ],
        out_specs=pl.BlockSpec((MAXLEN, blk, EMBED_DIM), lambda i: (0, i, 0)),
        out_shape=jax.ShapeDtypeStruct((MAXLEN, n, EMBED_DIM), x.dtype),
        compiler_params=pltpu.CompilerParams(
            dimension_semantics=("arbitrary",),
            vmem_limit_bytes=100 * 1024 * 1024,
        ),
    )(xt, pos_table)
    return jnp.transpose(out, (1, 0, 2))
